# baseline (speedup 1.0000x reference)
import jax
import jax.numpy as jnp
from jax import lax
from jax.experimental import pallas as pl
from jax.experimental.pallas import tpu as pltpu
from jax.experimental.pallas import tpu_sc as plsc

_NEG = -1e30
_SPLIT = 1024
_NC = 2
_NS = 16
_CHUNK = 8


def _target_kernel(h_ref, w_ref, out_ref):
    out_ref[...] = jax.lax.dot_general(
        h_ref[...], w_ref[...], (((1,), (1,)), ((), ())),
        preferred_element_type=jnp.float32)


def _attn_kernel(lens_ref, sel_ref, ctx_ref, tgt_ref, attn_ref, wout_ref,
                 scl_ref, md_ref, acc_ref):
    b = pl.program_id(0)
    s = pl.program_id(1)
    ns = pl.num_programs(1)
    ts = ctx_ref.shape[1]
    nblk = lens_ref.shape[1]

    @pl.when(s == 0)
    def _init():
        sel = sel_ref[b]

        def body(j, tot):
            return tot + jnp.where(j < sel, lens_ref[b, j], 0)

        scl_ref[0] = jax.lax.fori_loop(0, nblk, body, 0) + 1
        scl_ref[1] = lens_ref[b, sel]
        md_ref[0] = _NEG
        md_ref[1] = 0.0
        acc_ref[...] = jnp.zeros_like(acc_ref)

    ctx = ctx_ref[0]
    tgt = tgt_ref[0]
    attn_row = jax.lax.dot_general(
        tgt, ctx, (((1,), (1,)), ((), ())),
        preferred_element_type=jnp.float32)
    attn_ref[0] = attn_row

    sc = scl_ref[0]
    ln = scl_ref[1]
    t0 = s * ts
    overlap = (t0 < sc + ln) & (t0 + ts > sc)

    @pl.when(overlap)
    def _update():
        pos = t0 + jax.lax.broadcasted_iota(jnp.int32, (1, ts), 1)
        mask = (pos >= sc) & (pos < sc + ln)
        masked = jnp.where(mask, attn_row, _NEG)
        m_old = md_ref[0]
        m_new = jnp.maximum(m_old, jnp.max(masked))
        scale = jnp.exp(m_old - m_new)
        unnorm = jnp.where(mask, jnp.exp(masked - m_new), 0.0)
        md_ref[0] = m_new
        md_ref[1] = md_ref[1] * scale + jnp.sum(unnorm)
        acc_ref[...] = acc_ref[...] * scale + jax.lax.dot_general(
            unnorm, ctx, (((1,), (0,)), ((), ())),
            preferred_element_type=jnp.float32)

    @pl.when(s == ns - 1)
    def _finalize():
        d = md_ref[1]
        denom = jnp.where(d == 0.0, 1.0, d)
        wout_ref[0] = acc_ref[...] / denom


def _sc_attn_tail(ctx_hbm, tgt_hbm, out_hbm, tgt_v, buf0, buf1, out_v,
                  sem0, sem1):
    dim = tgt_v.shape[0]
    batch = ctx_hbm.shape[0]
    seq = ctx_hbm.shape[1]
    tail = seq - _SPLIT
    nw = _NC * _NS
    tiles_per_batch = nw // batch
    rows_per_tile = tail // tiles_per_batch

    cid = lax.axis_index("c")
    sid = lax.axis_index("s")
    wid = sid * _NC + cid
    b = wid // tiles_per_batch
    k = wid % tiles_per_batch
    row0 = _SPLIT + k * rows_per_tile

    pltpu.sync_copy(tgt_hbm.at[b], tgt_v)

    nchunks = rows_per_tile // _CHUNK
    ngroups = nchunks // 2

    def _issue(chunk, buf, sem):
        pltpu.async_copy(
            ctx_hbm.at[b, pl.ds(row0 + chunk * _CHUNK, _CHUNK)], buf, sem)

    def _wait(buf, sem):
        pltpu.make_async_copy(
            ctx_hbm.at[b, pl.ds(row0, _CHUNK)], buf, sem).wait()

    def _dots(buf, lane_base):
        def body(j, accs):
            tc = tgt_v[pl.ds(j * 16, 16)]
            return tuple(accs[r] + buf[r, pl.ds(j * 16, 16)] * tc
                         for r in range(_CHUNK))

        accs = lax.fori_loop(
            0, dim // 16, body,
            tuple(jnp.zeros((16,), jnp.float32) for _ in range(_CHUNK)))
        lanes = lax.iota(jnp.int32, 16)
        sums = jnp.zeros((16,), jnp.float32)
        for r in range(_CHUNK):
            sums = jnp.where(lanes == lane_base + r, jnp.sum(accs[r]), sums)
        return sums

    _issue(0, buf0, sem0)
    _issue(1, buf1, sem1)

    def group(g, carry):
        _wait(buf0, sem0)
        s0 = _dots(buf0, 0)

        @pl.when(g + 1 < ngroups)
        def _():
            _issue(2 * g + 2, buf0, sem0)

        _wait(buf1, sem1)
        s1 = _dots(buf1, 8)

        @pl.when(g + 1 < ngroups)
        def _():
            _issue(2 * g + 3, buf1, sem1)

        out_v[pl.ds(g * 16, 16)] = s0 + s1
        return carry

    lax.fori_loop(0, ngroups, group, 0)
    pltpu.sync_copy(out_v, out_hbm.at[b, pl.ds(k * rows_per_tile,
                                               rows_per_tile)])


def kernel(h, context, sub_seq_lengths, selected_block_idx, W):
    batch, dim = h.shape
    seq = context.shape[1]

    td = 1024
    target = pl.pallas_call(
        _target_kernel,
        grid=(dim // td,),
        in_specs=[
            pl.BlockSpec((batch, dim), lambda i: (0, 0)),
            pl.BlockSpec((td, dim), lambda i: (i, 0)),
        ],
        out_specs=pl.BlockSpec((batch, td), lambda i: (0, i)),
        out_shape=jax.ShapeDtypeStruct((batch, dim), jnp.float32),
    )(h, W)

    tail = seq - _SPLIT
    attn_tail = pl.kernel(
        _sc_attn_tail,
        out_type=jax.ShapeDtypeStruct((batch, tail), jnp.float32),
        mesh=plsc.VectorSubcoreMesh(core_axis_name="c", subcore_axis_name="s"),
        compiler_params=pltpu.CompilerParams(needs_layout_passes=False),
        scratch_types=[
            pltpu.VMEM((dim,), jnp.float32),
            pltpu.VMEM((_CHUNK, dim), jnp.float32),
            pltpu.VMEM((_CHUNK, dim), jnp.float32),
            pltpu.VMEM((tail * batch // (_NC * _NS),), jnp.float32),
            pltpu.SemaphoreType.DMA,
            pltpu.SemaphoreType.DMA,
        ],
    )(context, target)

    ts = 1024
    ns = _SPLIT // ts
    lens = sub_seq_lengths.astype(jnp.int32)
    sel = selected_block_idx.astype(jnp.int32)
    tgt3 = target.reshape(batch, 1, dim)
    attn_head, weighted = pl.pallas_call(
        _attn_kernel,
        grid=(batch, ns),
        in_specs=[
            pl.BlockSpec(memory_space=pltpu.SMEM),
            pl.BlockSpec(memory_space=pltpu.SMEM),
            pl.BlockSpec((1, ts, dim), lambda b, s: (b, s, 0)),
            pl.BlockSpec((1, 1, dim), lambda b, s: (b, 0, 0)),
        ],
        out_specs=[
            pl.BlockSpec((1, 1, ts), lambda b, s: (b * ns + s, 0, 0)),
            pl.BlockSpec((1, 1, dim), lambda b, s: (b, 0, 0)),
        ],
        out_shape=[
            jax.ShapeDtypeStruct((batch * ns, 1, ts), jnp.float32),
            jax.ShapeDtypeStruct((batch, 1, dim), jnp.float32),
        ],
        scratch_shapes=[
            pltpu.SMEM((2,), jnp.int32),
            pltpu.SMEM((2,), jnp.float32),
            pltpu.VMEM((1, dim), jnp.float32),
        ],
    )(lens, sel, context, tgt3)
    attn = jnp.concatenate(
        [attn_head.reshape(batch, _SPLIT), attn_tail], axis=1)
    return (weighted.reshape(batch, dim), attn)


# speedup vs baseline: 1.2659x; 1.2659x over previous
import jax
import jax.numpy as jnp
from jax.experimental import pallas as pl
from jax.experimental.pallas import tpu as pltpu

_NEG = -1e30
_TD = 512
_TS = 1024


def _fused_kernel(lens_ref, sel_ref, h_ref, w_ref, ctx_ref,
                  attn_ref, wout_ref, tgt_ref, scl_ref, md_ref, acc_ref):
    i = pl.program_id(0)
    dim = h_ref.shape[1]
    nw = dim // _TD
    ts = ctx_ref.shape[1]
    nblk = lens_ref.shape[1]
    nsteps = pl.num_programs(0)

    @pl.when(i < nw)
    def _w_phase():
        tgt_ref[:, pl.ds(i * _TD, _TD)] = jax.lax.dot_general(
            h_ref[...], w_ref[...], (((1,), (1,)), ((), ())),
            preferred_element_type=jnp.float32)

    @pl.when(i >= nw)
    def _ctx_phase():
        j = i - nw
        nctx = nsteps - nw
        batch = h_ref.shape[0]
        ns = nctx // batch
        b = j // ns
        s = j % ns

        @pl.when(s == 0)
        def _init():
            sel = sel_ref[b]

            def body(k, tot):
                return tot + jnp.where(k < sel, lens_ref[b, k], 0)

            scl_ref[0] = jax.lax.fori_loop(0, nblk, body, 0) + 1
            scl_ref[1] = lens_ref[b, sel]
            md_ref[0] = _NEG
            md_ref[1] = 0.0
            acc_ref[...] = jnp.zeros_like(acc_ref)

        ctx = ctx_ref[0]
        tgt = tgt_ref[pl.ds(b, 1), :]
        attn_row = jax.lax.dot_general(
            tgt, ctx, (((1,), (1,)), ((), ())),
            preferred_element_type=jnp.float32)
        attn_ref[0] = attn_row

        sc = scl_ref[0]
        ln = scl_ref[1]
        t0 = s * ts
        overlap = (t0 < sc + ln) & (t0 + ts > sc)

        @pl.when(overlap)
        def _update():
            pos = t0 + jax.lax.broadcasted_iota(jnp.int32, (1, ts), 1)
            mask = (pos >= sc) & (pos < sc + ln)
            masked = jnp.where(mask, attn_row, _NEG)
            m_old = md_ref[0]
            m_new = jnp.maximum(m_old, jnp.max(masked))
            scale = jnp.exp(m_old - m_new)
            unnorm = jnp.where(mask, jnp.exp(masked - m_new), 0.0)
            md_ref[0] = m_new
            md_ref[1] = md_ref[1] * scale + jnp.sum(unnorm)
            acc_ref[...] = acc_ref[...] * scale + jax.lax.dot_general(
                unnorm, ctx, (((1,), (0,)), ((), ())),
                preferred_element_type=jnp.float32)

        @pl.when(s == ns - 1)
        def _finalize():
            d = md_ref[1]
            denom = jnp.where(d == 0.0, 1.0, d)
            wout_ref[0] = acc_ref[...] / denom


def kernel(h, context, sub_seq_lengths, selected_block_idx, W):
    batch, dim = h.shape
    seq = context.shape[1]
    nw = dim // _TD
    ns = seq // _TS
    nctx = batch * ns
    lens = sub_seq_lengths.astype(jnp.int32)
    sel = selected_block_idx.astype(jnp.int32)

    attn, weighted = pl.pallas_call(
        _fused_kernel,
        grid=(nw + nctx,),
        in_specs=[
            pl.BlockSpec(memory_space=pltpu.SMEM),
            pl.BlockSpec(memory_space=pltpu.SMEM),
            pl.BlockSpec((batch, dim), lambda i: (0, 0)),
            pl.BlockSpec((_TD, dim), lambda i: (jnp.minimum(i, nw - 1), 0)),
            pl.BlockSpec(
                (1, _TS, dim),
                lambda i: ((jnp.maximum(i - nw, 0)) // ns,
                           (jnp.maximum(i - nw, 0)) % ns, 0)),
        ],
        out_specs=[
            pl.BlockSpec((1, 1, _TS),
                         lambda i: (jnp.maximum(i - nw, 0), 0, 0)),
            pl.BlockSpec((1, 1, dim),
                         lambda i: ((jnp.maximum(i - nw, 0)) // ns, 0, 0)),
        ],
        out_shape=[
            jax.ShapeDtypeStruct((nctx, 1, _TS), jnp.float32),
            jax.ShapeDtypeStruct((batch, 1, dim), jnp.float32),
        ],
        scratch_shapes=[
            pltpu.VMEM((batch, dim), jnp.float32),
            pltpu.SMEM((2,), jnp.int32),
            pltpu.SMEM((2,), jnp.float32),
            pltpu.VMEM((1, dim), jnp.float32),
        ],
    )(lens, sel, h, W, context)
    return (weighted.reshape(batch, dim), attn.reshape(batch, seq))


# speedup vs baseline: 1.2675x; 1.0013x over previous
import jax
import jax.numpy as jnp
from jax.experimental import pallas as pl
from jax.experimental.pallas import tpu as pltpu

_NEG = -1e30
_TD = 512
_TS = 1024


def _fused_kernel(lens_ref, sel_ref, h_ref, w_ref, ctx_ref,
                  attn_ref, wout_ref, tgt_ref, scl_ref, md_ref, acc_ref):
    i = pl.program_id(0)
    dim = h_ref.shape[1]
    nw = dim // _TD
    ts = ctx_ref.shape[1]
    nblk = lens_ref.shape[1]
    nsteps = pl.num_programs(0)

    @pl.when(i < nw)
    def _w_phase():
        tgt_ref[:, pl.ds(i * _TD, _TD)] = jax.lax.dot_general(
            h_ref[...], w_ref[...], (((1,), (1,)), ((), ())),
            preferred_element_type=jnp.float32)

    @pl.when(i >= nw)
    def _ctx_phase():
        j = i - nw
        nctx = nsteps - nw
        batch = h_ref.shape[0]
        ns = nctx // batch
        b = j // ns
        s = j % ns

        @pl.when(s == 0)
        def _init():
            sel = sel_ref[b]

            def body(k, tot):
                return tot + jnp.where(k < sel, lens_ref[b, k], 0)

            scl_ref[0] = jax.lax.fori_loop(0, nblk, body, 0) + 1
            scl_ref[1] = lens_ref[b, sel]
            md_ref[0] = _NEG
            md_ref[1] = 0.0
            acc_ref[...] = jnp.zeros_like(acc_ref)

        ctx = ctx_ref[0]
        tgt = tgt_ref[pl.ds(b, 1), :]
        attn_row = jax.lax.dot_general(
            tgt, ctx, (((1,), (1,)), ((), ())),
            preferred_element_type=jnp.float32)
        attn_ref[0, :, pl.ds(s * ts, ts)] = attn_row

        sc = scl_ref[0]
        ln = scl_ref[1]
        t0 = s * ts
        overlap = (t0 < sc + ln) & (t0 + ts > sc)

        @pl.when(overlap)
        def _update():
            pos = t0 + jax.lax.broadcasted_iota(jnp.int32, (1, ts), 1)
            mask = (pos >= sc) & (pos < sc + ln)
            masked = jnp.where(mask, attn_row, _NEG)
            m_old = md_ref[0]
            m_new = jnp.maximum(m_old, jnp.max(masked))
            scale = jnp.exp(m_old - m_new)
            unnorm = jnp.where(mask, jnp.exp(masked - m_new), 0.0)
            md_ref[0] = m_new
            md_ref[1] = md_ref[1] * scale + jnp.sum(unnorm)
            acc_ref[...] = acc_ref[...] * scale + jax.lax.dot_general(
                unnorm, ctx, (((1,), (0,)), ((), ())),
                preferred_element_type=jnp.float32)

        @pl.when(s == ns - 1)
        def _finalize():
            d = md_ref[1]
            denom = jnp.where(d == 0.0, 1.0, d)
            wout_ref[0] = acc_ref[...] / denom


def kernel(h, context, sub_seq_lengths, selected_block_idx, W):
    batch, dim = h.shape
    seq = context.shape[1]
    nw = dim // _TD
    ns = seq // _TS
    nctx = batch * ns
    lens = sub_seq_lengths.astype(jnp.int32)
    sel = selected_block_idx.astype(jnp.int32)

    attn, weighted = pl.pallas_call(
        _fused_kernel,
        grid=(nw + nctx,),
        in_specs=[
            pl.BlockSpec(memory_space=pltpu.SMEM),
            pl.BlockSpec(memory_space=pltpu.SMEM),
            pl.BlockSpec((batch, dim), lambda i: (0, 0)),
            pl.BlockSpec((_TD, dim), lambda i: (jnp.minimum(i, nw - 1), 0)),
            pl.BlockSpec(
                (1, _TS, dim),
                lambda i: ((jnp.maximum(i - nw, 0)) // ns,
                           (jnp.maximum(i - nw, 0)) % ns, 0)),
        ],
        out_specs=[
            pl.BlockSpec((1, 1, seq),
                         lambda i: ((jnp.maximum(i - nw, 0)) // ns, 0, 0)),
            pl.BlockSpec((1, 1, dim),
                         lambda i: ((jnp.maximum(i - nw, 0)) // ns, 0, 0)),
        ],
        out_shape=[
            jax.ShapeDtypeStruct((batch, 1, seq), jnp.float32),
            jax.ShapeDtypeStruct((batch, 1, dim), jnp.float32),
        ],
        scratch_shapes=[
            pltpu.VMEM((batch, dim), jnp.float32),
            pltpu.SMEM((2,), jnp.int32),
            pltpu.SMEM((2,), jnp.float32),
            pltpu.VMEM((1, dim), jnp.float32),
        ],
    )(lens, sel, h, W, context)
    return (weighted.reshape(batch, dim), attn.reshape(batch, seq))
